# baseline (speedup 1.0000x reference)
import jax
import jax.numpy as jnp
from jax.experimental import pallas as pl
from jax.experimental.pallas import tpu as pltpu

N = 4096
D = 64
BM = 256


def _matmul_block(inp_ref, w_ref, out_ref):
    out_ref[...] = jnp.dot(inp_ref[...], w_ref[...],
                           preferred_element_type=jnp.float32)


@jax.jit
def kernel(inp, weight):
    grid = (N // BM,)
    return pl.pallas_call(
        _matmul_block,
        grid=grid,
        in_specs=[
            pl.BlockSpec((BM, N), lambda i: (i, 0)),
            pl.BlockSpec((N, D), lambda i: (0, 0)),
        ],
        out_specs=pl.BlockSpec((BM, D), lambda i: (i, 0)),
        out_shape=jax.ShapeDtypeStruct((N, D), jnp.float32),
        compiler_params=pltpu.CompilerParams(
            skip_device_barrier=True,
            disable_bounds_checks=True,
        ),
    )(inp, weight)


# speedup vs baseline: 1.1876x; 1.1876x over previous
import jax
import jax.numpy as jnp
from jax.experimental import pallas as pl
from jax.experimental.pallas import tpu as pltpu

N = 4096
D = 64
BM = 512


def _matmul_block(inp_ref, w_ref, out_ref):
    out_ref[...] = jnp.dot(inp_ref[...], w_ref[...],
                           preferred_element_type=jnp.float32)


@jax.jit
def kernel(inp, weight):
    grid = (N // BM,)
    return pl.pallas_call(
        _matmul_block,
        grid=grid,
        in_specs=[
            pl.BlockSpec((BM, N), lambda i: (i, 0)),
            pl.BlockSpec((N, D), lambda i: (0, 0)),
        ],
        out_specs=pl.BlockSpec((BM, D), lambda i: (i, 0)),
        out_shape=jax.ShapeDtypeStruct((N, D), jnp.float32),
        compiler_params=pltpu.CompilerParams(
            skip_device_barrier=True,
            disable_bounds_checks=True,
        ),
    )(inp, weight)
